# Initial kernel scaffold; baseline (speedup 1.0000x reference)
import jax
import jax.numpy as jnp
from jax.experimental import pallas as pl


def kernel(node_features, edge_features, senders, receivers, edge_params, node_params):
    raise NotImplementedError("write your pallas kernel here")


# speedup vs baseline: 3.5197x; 3.5197x over previous
import functools

import jax
import jax.numpy as jnp
import numpy as np
from jax import lax
from jax.experimental import pallas as pl
from jax.experimental.pallas import tpu as pltpu
from jax.experimental.pallas import tpu_sc as plsc

NC = 2
NS = 16
NW = NC * NS
ROW = 128


def _worker_split(total_rows):
    base = total_rows // NW
    extra = total_rows % NW
    return base, extra


def _worker_slabs(idx_rows, total_rows):
    base, extra = _worker_split(total_rows)
    rpw8 = -(-(base + 1) // 8) * 8
    starts = np.array([w * base + min(w, extra) for w in range(NW)])
    rows = np.minimum(starts[:, None] + np.arange(rpw8)[None, :], total_rows - 1)
    return jnp.take(idx_rows, jnp.asarray(rows), axis=0), rpw8


def _sc_gather(table, idx_slabs, rpw8, total_rows):
    n, d = table.shape
    base, extra = _worker_split(total_rows)
    mesh = plsc.VectorSubcoreMesh(
        core_axis_name="c", subcore_axis_name="s", num_cores=NC, num_subcores=NS
    )

    @functools.partial(
        pl.kernel,
        out_type=jax.ShapeDtypeStruct((total_rows * ROW, d), jnp.float32),
        mesh=mesh,
        scratch_types=[
            pltpu.VMEM((rpw8, ROW), jnp.int32),
            pltpu.VMEM((ROW, d), jnp.float32),
            pltpu.SemaphoreType.DMA,
        ],
    )
    def k(table_hbm, idx_hbm, out_hbm, idx_v, buf, gsem):
        c = lax.axis_index("c")
        s = lax.axis_index("s")
        w = s * NC + c
        start = w * base + jnp.minimum(w, extra)
        cnt = base + jnp.where(w < extra, 1, 0)
        pltpu.sync_copy(idx_hbm.at[w], idx_v)

        def body(j, _):
            pltpu.async_copy(table_hbm.at[idx_v.at[j]], buf, gsem).wait()
            pltpu.sync_copy(buf, out_hbm.at[pl.ds((start + j) * ROW, ROW)])
            return 0

        lax.fori_loop(0, cnt, body, 0)

    return k(table, idx_slabs)


def _ln_affine(h, gamma, beta, eps=1e-5):
    mu = jnp.mean(h, axis=-1, keepdims=True)
    xc = h - mu
    var = jnp.mean(xc * xc, axis=-1, keepdims=True)
    return xc * lax.rsqrt(var + eps) * gamma + beta


def _edge_mlp_body(gs_ref, gr_ref, e_ref, w0_ref, b0_ref, w1_ref, b1_ref,
                   w2_ref, b2_ref, g_ref, bt_ref, mlp_ref, edge_ref):
    s = gs_ref[0]
    r = gr_ref[0]
    e = e_ref[...]
    w0 = w0_ref[...]
    d = e.shape[1]
    h = (jnp.dot(s, w0[0:d], preferred_element_type=jnp.float32)
         + jnp.dot(r, w0[d:2 * d], preferred_element_type=jnp.float32)
         + jnp.dot(e, w0[2 * d:3 * d], preferred_element_type=jnp.float32)
         + b0_ref[...])
    h = jnp.maximum(h, 0.0)
    h = jnp.maximum(jnp.dot(h, w1_ref[...], preferred_element_type=jnp.float32)
                    + b1_ref[...], 0.0)
    h = jnp.dot(h, w2_ref[...], preferred_element_type=jnp.float32) + b2_ref[...]
    h = _ln_affine(h, g_ref[...], bt_ref[...])
    mlp_ref[...] = h
    edge_ref[...] = h + e


def _tc_edge_mlp(gathered, edge_features, edge_params, block_e):
    e_total, d = edge_features.shape
    nb = e_total // block_e
    w0, b0, w1, b1, w2, b2, gamma, beta = edge_params
    vec = lambda v: v.reshape(1, d)
    g3 = gathered.reshape(2, e_total, d)

    out = pl.pallas_call(
        _edge_mlp_body,
        grid=(nb,),
        in_specs=[
            pl.BlockSpec((1, block_e, d), lambda i: (0, i, 0)),
            pl.BlockSpec((1, block_e, d), lambda i: (1, i, 0)),
            pl.BlockSpec((block_e, d), lambda i: (i, 0)),
            pl.BlockSpec((3 * d, d), lambda i: (0, 0)),
            pl.BlockSpec((1, d), lambda i: (0, 0)),
            pl.BlockSpec((d, d), lambda i: (0, 0)),
            pl.BlockSpec((1, d), lambda i: (0, 0)),
            pl.BlockSpec((d, d), lambda i: (0, 0)),
            pl.BlockSpec((1, d), lambda i: (0, 0)),
            pl.BlockSpec((1, d), lambda i: (0, 0)),
            pl.BlockSpec((1, d), lambda i: (0, 0)),
        ],
        out_specs=[
            pl.BlockSpec((block_e, d), lambda i: (i, 0)),
            pl.BlockSpec((block_e, d), lambda i: (i, 0)),
        ],
        out_shape=[
            jax.ShapeDtypeStruct((e_total, d), jnp.float32),
            jax.ShapeDtypeStruct((e_total, d), jnp.float32),
        ],
        compiler_params=pltpu.CompilerParams(
            dimension_semantics=("arbitrary",),
        ),
    )(g3, g3, edge_features, w0, vec(b0), w1, vec(b1), w2, vec(b2),
      vec(gamma), vec(beta))
    return out


def _sc_scatter(values, idx_slabs, rpw8, n, total_rows):
    d = values.shape[1]
    base, extra = _worker_split(total_rows)
    stride = (n // NS) // 8 * 8
    span = n - (NS - 1) * stride
    assert span % 8 == 0 and span >= stride
    mesh = plsc.VectorSubcoreMesh(
        core_axis_name="c", subcore_axis_name="s", num_cores=NC, num_subcores=NS
    )

    @functools.partial(
        pl.kernel,
        out_type=jax.ShapeDtypeStruct((NC, n, d), jnp.float32),
        mesh=mesh,
        scratch_types=[
            pltpu.VMEM((rpw8, ROW), jnp.int32),
            pltpu.VMEM((ROW, d), jnp.float32),
            pltpu.VMEM_SHARED((n, d), jnp.float32),
            pltpu.SemaphoreType.DMA,
        ],
    )
    def k(val_hbm, idx_hbm, zero_hbm, out_hbm, idx_v, buf, acc, sem):
        c = lax.axis_index("c")
        s = lax.axis_index("s")
        w = s * NC + c
        start = w * base + jnp.minimum(w, extra)
        cnt = base + jnp.where(w < extra, 1, 0)

        pltpu.sync_copy(zero_hbm.at[pl.ds(s * stride, span)], acc.at[pl.ds(s * stride, span)])
        pltpu.sync_copy(idx_hbm.at[w], idx_v)
        plsc.subcore_barrier()

        def body(j, _):
            pltpu.async_copy(val_hbm.at[pl.ds((start + j) * ROW, ROW)], buf, sem).wait()
            pltpu.sync_copy(buf, acc.at[idx_v.at[j]], add=True)
            return 0

        lax.fori_loop(0, cnt, body, 0)
        plsc.subcore_barrier()
        pltpu.sync_copy(acc.at[pl.ds(s * stride, span)], out_hbm.at[c, pl.ds(s * stride, span)])

    zeros = jnp.zeros((n, d), jnp.float32)
    return k(values, idx_slabs, zeros)


def _node_mlp_body(nf_ref, p0_ref, p1_ref, w0_ref, b0_ref, w1_ref, b1_ref,
                   w2_ref, b2_ref, g_ref, bt_ref, out_ref):
    nf = nf_ref[...]
    seg = p0_ref[0] + p1_ref[0]
    w0 = w0_ref[...]
    d = nf.shape[1]
    h = (jnp.dot(nf, w0[0:d], preferred_element_type=jnp.float32)
         + jnp.dot(seg, w0[d:2 * d], preferred_element_type=jnp.float32)
         + b0_ref[...])
    h = jnp.maximum(h, 0.0)
    h = jnp.maximum(jnp.dot(h, w1_ref[...], preferred_element_type=jnp.float32)
                    + b1_ref[...], 0.0)
    h = jnp.dot(h, w2_ref[...], preferred_element_type=jnp.float32) + b2_ref[...]
    h = _ln_affine(h, g_ref[...], bt_ref[...])
    out_ref[...] = h + nf


def _tc_node_mlp(node_features, partials, node_params, block_n):
    n, d = node_features.shape
    nb = n // block_n
    w0, b0, w1, b1, w2, b2, gamma, beta = node_params
    vec = lambda v: v.reshape(1, d)

    return pl.pallas_call(
        _node_mlp_body,
        grid=(nb,),
        in_specs=[
            pl.BlockSpec((block_n, d), lambda i: (i, 0)),
            pl.BlockSpec((1, block_n, d), lambda i: (0, i, 0)),
            pl.BlockSpec((1, block_n, d), lambda i: (1, i, 0)),
            pl.BlockSpec((2 * d, d), lambda i: (0, 0)),
            pl.BlockSpec((1, d), lambda i: (0, 0)),
            pl.BlockSpec((d, d), lambda i: (0, 0)),
            pl.BlockSpec((1, d), lambda i: (0, 0)),
            pl.BlockSpec((d, d), lambda i: (0, 0)),
            pl.BlockSpec((1, d), lambda i: (0, 0)),
            pl.BlockSpec((1, d), lambda i: (0, 0)),
            pl.BlockSpec((1, d), lambda i: (0, 0)),
        ],
        out_specs=pl.BlockSpec((block_n, d), lambda i: (i, 0)),
        out_shape=jax.ShapeDtypeStruct((n, d), jnp.float32),
        compiler_params=pltpu.CompilerParams(
            dimension_semantics=("arbitrary",),
        ),
    )(node_features, partials, partials, w0, vec(b0), w1, vec(b1), w2, vec(b2),
      vec(gamma), vec(beta))


def kernel(node_features, edge_features, senders, receivers, edge_params, node_params):
    n, d = node_features.shape
    e = edge_features.shape[0]
    assert e % ROW == 0 and n % NS == 0

    r_gather = 2 * e // ROW
    idx_all = jnp.concatenate([senders, receivers]).reshape(r_gather, ROW)
    g_slabs, g_rpw8 = _worker_slabs(idx_all, r_gather)
    gathered = _sc_gather(node_features, g_slabs, g_rpw8, r_gather)

    mlp_out, new_edge = _tc_edge_mlp(gathered, edge_features, edge_params,
                                     block_e=2000)

    r_scatter = e // ROW
    s_slabs, s_rpw8 = _worker_slabs(receivers.reshape(r_scatter, ROW), r_scatter)
    partials = _sc_scatter(mlp_out, s_slabs, s_rpw8, n, r_scatter)

    new_node = _tc_node_mlp(node_features, partials, node_params, block_n=1000)
    return (new_node, new_edge)


# speedup vs baseline: 4.4155x; 1.2545x over previous
import functools

import jax
import jax.numpy as jnp
import numpy as np
from jax import lax
from jax.experimental import pallas as pl
from jax.experimental.pallas import tpu as pltpu
from jax.experimental.pallas import tpu_sc as plsc

NC = 2
NS = 16
NW = NC * NS
ROW = 128


def _worker_split(total_rows):
    base = total_rows // NW
    extra = total_rows % NW
    return base, extra


def _worker_slabs(idx_rows, total_rows):
    base, extra = _worker_split(total_rows)
    rpw8 = -(-(base + 1) // 8) * 8
    starts = np.array([w * base + min(w, extra) for w in range(NW)])
    rows = np.minimum(starts[:, None] + np.arange(rpw8)[None, :], total_rows - 1)
    return jnp.take(idx_rows, jnp.asarray(rows), axis=0), rpw8


def _sc_gather(table, idx_slabs, rpw8, total_rows):
    n, d = table.shape
    base, extra = _worker_split(total_rows)
    mesh = plsc.VectorSubcoreMesh(
        core_axis_name="c", subcore_axis_name="s", num_cores=NC, num_subcores=NS
    )

    nbuf = 4

    @functools.partial(
        pl.kernel,
        out_type=jax.ShapeDtypeStruct((total_rows * ROW, d), jnp.float32),
        mesh=mesh,
        scratch_types=[
            pltpu.VMEM((rpw8, ROW), jnp.int32),
            [pltpu.VMEM((ROW, d), jnp.float32) for _ in range(nbuf)],
            [pltpu.SemaphoreType.DMA for _ in range(nbuf)],
        ],
    )
    def k(table_hbm, idx_hbm, out_hbm, idx_v, bufs, gsems):
        c = lax.axis_index("c")
        s = lax.axis_index("s")
        w = s * NC + c
        start = w * base + jnp.minimum(w, extra)
        cnt = base + jnp.where(w < extra, 1, 0)
        pltpu.sync_copy(idx_hbm.at[w], idx_v)

        for p in range(nbuf):
            @pl.when(p < cnt)
            def _():
                pltpu.async_copy(table_hbm.at[idx_v.at[p]], bufs[p], gsems[p])

        def body(j, _):
            for p in range(nbuf):
                @pl.when(lax.rem(j, nbuf) == p)
                def _():
                    pltpu.make_async_copy(
                        table_hbm.at[idx_v.at[0]], bufs[p], gsems[p]).wait()
                    pltpu.sync_copy(bufs[p], out_hbm.at[pl.ds((start + j) * ROW, ROW)])

                    @pl.when(j + nbuf < cnt)
                    def _():
                        pltpu.async_copy(table_hbm.at[idx_v.at[j + nbuf]],
                                         bufs[p], gsems[p])
            return 0

        lax.fori_loop(0, cnt, body, 0)

    return k(table, idx_slabs)


def _ln_affine(h, gamma, beta, eps=1e-5):
    mu = jnp.mean(h, axis=-1, keepdims=True)
    xc = h - mu
    var = jnp.mean(xc * xc, axis=-1, keepdims=True)
    return xc * lax.rsqrt(var + eps) * gamma + beta


def _edge_mlp_body(gs_ref, gr_ref, e_ref, w0_ref, b0_ref, w1_ref, b1_ref,
                   w2_ref, b2_ref, g_ref, bt_ref, mlp_ref, edge_ref):
    s = gs_ref[0]
    r = gr_ref[0]
    e = e_ref[...]
    w0 = w0_ref[...]
    d = e.shape[1]
    h = (jnp.dot(s, w0[0:d], preferred_element_type=jnp.float32)
         + jnp.dot(r, w0[d:2 * d], preferred_element_type=jnp.float32)
         + jnp.dot(e, w0[2 * d:3 * d], preferred_element_type=jnp.float32)
         + b0_ref[...])
    h = jnp.maximum(h, 0.0)
    h = jnp.maximum(jnp.dot(h, w1_ref[...], preferred_element_type=jnp.float32)
                    + b1_ref[...], 0.0)
    h = jnp.dot(h, w2_ref[...], preferred_element_type=jnp.float32) + b2_ref[...]
    h = _ln_affine(h, g_ref[...], bt_ref[...])
    mlp_ref[...] = h
    edge_ref[...] = h + e


def _tc_edge_mlp(gathered, edge_features, edge_params, block_e):
    e_total, d = edge_features.shape
    nb = e_total // block_e
    w0, b0, w1, b1, w2, b2, gamma, beta = edge_params
    vec = lambda v: v.reshape(1, d)
    g3 = gathered.reshape(2, e_total, d)

    out = pl.pallas_call(
        _edge_mlp_body,
        grid=(nb,),
        in_specs=[
            pl.BlockSpec((1, block_e, d), lambda i: (0, i, 0)),
            pl.BlockSpec((1, block_e, d), lambda i: (1, i, 0)),
            pl.BlockSpec((block_e, d), lambda i: (i, 0)),
            pl.BlockSpec((3 * d, d), lambda i: (0, 0)),
            pl.BlockSpec((1, d), lambda i: (0, 0)),
            pl.BlockSpec((d, d), lambda i: (0, 0)),
            pl.BlockSpec((1, d), lambda i: (0, 0)),
            pl.BlockSpec((d, d), lambda i: (0, 0)),
            pl.BlockSpec((1, d), lambda i: (0, 0)),
            pl.BlockSpec((1, d), lambda i: (0, 0)),
            pl.BlockSpec((1, d), lambda i: (0, 0)),
        ],
        out_specs=[
            pl.BlockSpec((block_e, d), lambda i: (i, 0)),
            pl.BlockSpec((block_e, d), lambda i: (i, 0)),
        ],
        out_shape=[
            jax.ShapeDtypeStruct((e_total, d), jnp.float32),
            jax.ShapeDtypeStruct((e_total, d), jnp.float32),
        ],
        compiler_params=pltpu.CompilerParams(
            dimension_semantics=("arbitrary",),
        ),
    )(g3, g3, edge_features, w0, vec(b0), w1, vec(b1), w2, vec(b2),
      vec(gamma), vec(beta))
    return out


def _sc_scatter(values, idx_slabs, rpw8, n, total_rows):
    d = values.shape[1]
    base, extra = _worker_split(total_rows)
    stride = (n // NS) // 8 * 8
    span = n - (NS - 1) * stride
    assert span % 8 == 0 and span >= stride
    mesh = plsc.VectorSubcoreMesh(
        core_axis_name="c", subcore_axis_name="s", num_cores=NC, num_subcores=NS
    )

    @functools.partial(
        pl.kernel,
        out_type=jax.ShapeDtypeStruct((NC, n, d), jnp.float32),
        mesh=mesh,
        scratch_types=[
            pltpu.VMEM_SHARED((n, d), jnp.float32),
            pltpu.VMEM((rpw8, ROW), jnp.int32),
            [pltpu.VMEM((ROW, d), jnp.float32) for _ in range(2)],
            [pltpu.SemaphoreType.DMA for _ in range(2)],
        ],
    )
    def k(val_hbm, idx_hbm, zero_hbm, out_hbm, acc, idx_v, bufs, sems):
        nbuf = len(bufs)
        c = lax.axis_index("c")
        s = lax.axis_index("s")
        w = s * NC + c
        start = w * base + jnp.minimum(w, extra)
        cnt = base + jnp.where(w < extra, 1, 0)

        pltpu.sync_copy(zero_hbm.at[pl.ds(s * stride, span)], acc.at[pl.ds(s * stride, span)])
        pltpu.sync_copy(idx_hbm.at[w], idx_v)
        plsc.subcore_barrier()

        for p in range(nbuf):
            @pl.when(p < cnt)
            def _():
                pltpu.async_copy(val_hbm.at[pl.ds((start + p) * ROW, ROW)],
                                 bufs[p], sems[p])

        def body(j, _):
            for p in range(nbuf):
                @pl.when(lax.rem(j, nbuf) == p)
                def _():
                    pltpu.make_async_copy(
                        val_hbm.at[pl.ds(0, ROW)], bufs[p], sems[p]).wait()
                    pltpu.sync_copy(bufs[p], acc.at[idx_v.at[j]], add=True)

                    @pl.when(j + nbuf < cnt)
                    def _():
                        pltpu.async_copy(
                            val_hbm.at[pl.ds((start + j + nbuf) * ROW, ROW)],
                            bufs[p], sems[p])
            return 0

        lax.fori_loop(0, cnt, body, 0)
        plsc.subcore_barrier()
        pltpu.sync_copy(acc.at[pl.ds(s * stride, span)], out_hbm.at[c, pl.ds(s * stride, span)])

    zeros = jnp.zeros((n, d), jnp.float32)
    return k(values, idx_slabs, zeros)


def _node_mlp_body(nf_ref, p0_ref, p1_ref, w0_ref, b0_ref, w1_ref, b1_ref,
                   w2_ref, b2_ref, g_ref, bt_ref, out_ref):
    nf = nf_ref[...]
    seg = p0_ref[0] + p1_ref[0]
    w0 = w0_ref[...]
    d = nf.shape[1]
    h = (jnp.dot(nf, w0[0:d], preferred_element_type=jnp.float32)
         + jnp.dot(seg, w0[d:2 * d], preferred_element_type=jnp.float32)
         + b0_ref[...])
    h = jnp.maximum(h, 0.0)
    h = jnp.maximum(jnp.dot(h, w1_ref[...], preferred_element_type=jnp.float32)
                    + b1_ref[...], 0.0)
    h = jnp.dot(h, w2_ref[...], preferred_element_type=jnp.float32) + b2_ref[...]
    h = _ln_affine(h, g_ref[...], bt_ref[...])
    out_ref[...] = h + nf


def _tc_node_mlp(node_features, partials, node_params, block_n):
    n, d = node_features.shape
    nb = n // block_n
    w0, b0, w1, b1, w2, b2, gamma, beta = node_params
    vec = lambda v: v.reshape(1, d)

    return pl.pallas_call(
        _node_mlp_body,
        grid=(nb,),
        in_specs=[
            pl.BlockSpec((block_n, d), lambda i: (i, 0)),
            pl.BlockSpec((1, block_n, d), lambda i: (0, i, 0)),
            pl.BlockSpec((1, block_n, d), lambda i: (1, i, 0)),
            pl.BlockSpec((2 * d, d), lambda i: (0, 0)),
            pl.BlockSpec((1, d), lambda i: (0, 0)),
            pl.BlockSpec((d, d), lambda i: (0, 0)),
            pl.BlockSpec((1, d), lambda i: (0, 0)),
            pl.BlockSpec((d, d), lambda i: (0, 0)),
            pl.BlockSpec((1, d), lambda i: (0, 0)),
            pl.BlockSpec((1, d), lambda i: (0, 0)),
            pl.BlockSpec((1, d), lambda i: (0, 0)),
        ],
        out_specs=pl.BlockSpec((block_n, d), lambda i: (i, 0)),
        out_shape=jax.ShapeDtypeStruct((n, d), jnp.float32),
        compiler_params=pltpu.CompilerParams(
            dimension_semantics=("arbitrary",),
        ),
    )(node_features, partials, partials, w0, vec(b0), w1, vec(b1), w2, vec(b2),
      vec(gamma), vec(beta))


def kernel(node_features, edge_features, senders, receivers, edge_params, node_params):
    n, d = node_features.shape
    e = edge_features.shape[0]
    assert e % ROW == 0 and n % NS == 0

    r_gather = 2 * e // ROW
    idx_all = jnp.concatenate([senders, receivers]).reshape(r_gather, ROW)
    g_slabs, g_rpw8 = _worker_slabs(idx_all, r_gather)
    gathered = _sc_gather(node_features, g_slabs, g_rpw8, r_gather)

    mlp_out, new_edge = _tc_edge_mlp(gathered, edge_features, edge_params,
                                     block_e=2000)

    r_scatter = e // ROW
    s_slabs, s_rpw8 = _worker_slabs(receivers.reshape(r_scatter, ROW), r_scatter)
    partials = _sc_scatter(mlp_out, s_slabs, s_rpw8, n, r_scatter)

    new_node = _tc_node_mlp(node_features, partials, node_params, block_n=1000)
    return (new_node, new_edge)
